# Initial kernel scaffold; baseline (speedup 1.0000x reference)
import jax
import jax.numpy as jnp
from jax.experimental import pallas as pl


def kernel(positions, neighbor_list, W0, b0, W1, b1, W2, b2):
    raise NotImplementedError("write your pallas kernel here")


# speedup vs baseline: 3.7103x; 3.7103x over previous
import functools

import jax
import jax.numpy as jnp
import numpy as np
from jax import lax
from jax.experimental import pallas as pl
from jax.experimental.pallas import tpu as pltpu
from jax.experimental.pallas import tpu_sc as plsc

BOX = 1.0
SIGMA = 0.04
PRIOR_N = 12

_HALF = BOX / 2.0

_NW = 32
_SEG = 2000
_D = 4

_TILE = 4096


def _sc_gather_rows(table, idx, T, T_pad):
    chunk = T // _NW
    n_seg = chunk // _SEG
    mesh = plsc.VectorSubcoreMesh(core_axis_name="c", subcore_axis_name="s")

    @functools.partial(
        pl.kernel,
        mesh=mesh,
        out_type=jax.ShapeDtypeStruct((T_pad, _D), jnp.float32),
        scratch_types=[
            pltpu.VMEM((_SEG,), jnp.int32),
            pltpu.VMEM((_SEG, _D), jnp.float32),
            pltpu.SemaphoreType.DMA,
        ],
        compiler_params=pltpu.CompilerParams(use_tc_tiling_on_sc=False),
    )
    def k(tbl_hbm, idx_hbm, out_hbm, idx_v, rows_v, sem):
        wid = lax.axis_index("s") * 2 + lax.axis_index("c")
        base = wid * chunk
        for s in range(n_seg):
            seg = base + s * _SEG
            pltpu.sync_copy(idx_hbm.at[pl.ds(seg, _SEG)], idx_v)
            pltpu.async_copy(tbl_hbm.at[idx_v], rows_v, sem).wait()
            pltpu.sync_copy(rows_v, out_hbm.at[pl.ds(seg, _SEG)])

    return k(table, idx)


def _tc_body(T, K, rows_ref, pos_ref, w0t, w1t, w1, w0, w2, b0r, b1r,
             e_ref, a_ref, out_ref):
    i = pl.program_id(0)
    rows = rows_ref[...]
    cmat = jnp.eye(3, _D, dtype=jnp.float32)
    nbr = lax.dot_general(cmat, rows, (((1,), (1,)), ((), ())),
                          preferred_element_type=jnp.float32)
    src = jnp.dot(pos_ref[...], e_ref[...],
                  preferred_element_type=jnp.float32)
    dr = src - nbr
    dr = jnp.where(dr > _HALF, dr - BOX, dr)
    dr = jnp.where(dr < -_HALF, dr + BOX, dr)
    r2 = jnp.sum(dr * dr, axis=0, keepdims=True)
    invR = lax.rsqrt(r2)
    R = r2 * invR
    X = jnp.concatenate([dr, R, invR], axis=0)
    pre0 = jnp.dot(w0t[...], X, preferred_element_type=jnp.float32) + b0r[...]
    h0 = jnp.maximum(pre0, 0.0)
    pre1 = jnp.dot(w1t[...], h0, preferred_element_type=jnp.float32) + b1r[...]
    gp1 = jnp.where(pre1 > 0.0, w2[...], 0.0)
    gh0 = jnp.dot(w1[...], gp1, preferred_element_type=jnp.float32)
    gp0 = jnp.where(pre0 > 0.0, gh0, 0.0)
    gf = jnp.dot(w0[...], gp0, preferred_element_type=jnp.float32)
    t = SIGMA * invR
    t2 = t * t
    t4 = t2 * t2
    t12 = t4 * t4 * t4
    coef = (gf[3:4, :] * invR - gf[4:5, :] * (invR * invR * invR)
            - PRIOR_N * t12 * (invR * invR))
    force = -(gf[0:3, :] + coef * dr)
    tile = rows.shape[0]
    tok = lax.broadcasted_iota(jnp.int32, (1, tile), 1) + i * tile
    force = jnp.where(tok < T, force, 0.0)
    out_ref[...] = jnp.dot(force, a_ref[...], preferred_element_type=jnp.float32)


def kernel(positions, neighbor_list, W0, b0, W1, b1, W2, b2):
    B, N, _ = positions.shape
    K = neighbor_list.shape[2]
    T = B * N * K
    grid = pl.cdiv(T, _TILE)
    T_pad = grid * _TILE
    nodes_pad = T_pad // K
    npb = _TILE // K

    table = jnp.concatenate(
        [positions.reshape(B * N, 3),
         jnp.zeros((B * N, 1), jnp.float32)], axis=1)
    boff = (jnp.arange(B, dtype=jnp.int32) * N)[:, None, None]
    idx = (neighbor_list[..., 1] + boff).reshape(-1)

    rows = _sc_gather_rows(table, idx, T, T_pad)

    pos_pl = jnp.zeros((3, nodes_pad), jnp.float32)
    pos_pl = pos_pl.at[:, :B * N].set(positions.reshape(B * N, 3).T)

    e_exp = jnp.asarray(np.kron(np.eye(npb, dtype=np.float32),
                                np.ones((1, K), dtype=np.float32)))
    a_sum = jnp.asarray(np.kron(np.eye(npb, dtype=np.float32),
                                np.ones((K, 1), dtype=np.float32)))

    hid = W0.shape[1]
    out = pl.pallas_call(
        functools.partial(_tc_body, T, K),
        grid=(grid,),
        in_specs=[
            pl.BlockSpec((_TILE, _D), lambda i: (i, 0)),
            pl.BlockSpec((3, npb), lambda i: (0, i)),
            pl.BlockSpec((hid, W0.shape[0]), lambda i: (0, 0)),
            pl.BlockSpec((hid, hid), lambda i: (0, 0)),
            pl.BlockSpec((hid, hid), lambda i: (0, 0)),
            pl.BlockSpec((W0.shape[0], hid), lambda i: (0, 0)),
            pl.BlockSpec((hid, 1), lambda i: (0, 0)),
            pl.BlockSpec((hid, 1), lambda i: (0, 0)),
            pl.BlockSpec((hid, 1), lambda i: (0, 0)),
            pl.BlockSpec((npb, _TILE), lambda i: (0, 0)),
            pl.BlockSpec((_TILE, npb), lambda i: (0, 0)),
        ],
        out_specs=pl.BlockSpec((3, npb), lambda i: (0, i)),
        out_shape=jax.ShapeDtypeStruct((3, nodes_pad), jnp.float32),
    )(rows, pos_pl, W0.T, W1.T, W1, W0, W2, b0.reshape(hid, 1),
      b1.reshape(hid, 1), e_exp, a_sum)

    return out[:, :B * N].reshape(3, B, N).transpose(1, 2, 0)
